# baseline (speedup 1.0000x reference)
import functools

import jax
import jax.numpy as jnp
from jax import lax
from jax.experimental import pallas as pl
from jax.experimental.pallas import tpu as pltpu
from jax.experimental.pallas import tpu_sc as plsc

NUM_EXPERT = 8
EMBED_DIM = 768
FFN_DIM = 469
TILE = 128


@functools.lru_cache(maxsize=None)
def _sc_row_gather(n_rows_table, n_idx, dim):
    info = plsc.get_sparse_core_info()
    nc, ns = info.num_cores, info.num_subcores
    nw = nc * ns
    assert n_idx % nw == 0
    b_per_w = n_idx // nw
    chunk = 64
    assert b_per_w % chunk == 0
    n_chunks = b_per_w // chunk
    assert n_chunks >= 2
    mesh = plsc.VectorSubcoreMesh(core_axis_name="c", subcore_axis_name="s")

    @functools.partial(
        pl.kernel,
        mesh=mesh,
        out_type=jax.ShapeDtypeStruct((n_idx, dim), jnp.float32),
        scratch_types=[
            pltpu.VMEM((chunk,), jnp.int32),
            pltpu.VMEM((chunk,), jnp.int32),
            pltpu.VMEM((chunk, dim), jnp.float32),
            pltpu.VMEM((chunk, dim), jnp.float32),
            pltpu.SemaphoreType.DMA,
            pltpu.SemaphoreType.DMA,
            pltpu.SemaphoreType.DMA,
            pltpu.SemaphoreType.DMA,
        ],
    )
    def gather_k(table_hbm, idx_hbm, out_hbm, i0, i1, r0, r1, g0, g1, w0, w1):
        wid = lax.axis_index("s") * nc + lax.axis_index("c")
        base = wid * b_per_w
        idx_v = [i0, i1]
        rows_v = [r0, r1]
        gsem = [g0, g1]
        wsem = [w0, w1]
        gh = [None, None]
        wh = [None, None]
        for c in range(n_chunks):
            b = c & 1
            off = base + c * chunk
            if c >= 2:
                wh[b].wait()
            pltpu.sync_copy(idx_hbm.at[pl.ds(off, chunk)], idx_v[b])
            gh[b] = pltpu.async_copy(table_hbm.at[idx_v[b]], rows_v[b], gsem[b])
            if c >= 1:
                bp = (c - 1) & 1
                gh[bp].wait()
                wh[bp] = pltpu.async_copy(
                    rows_v[bp],
                    out_hbm.at[pl.ds(base + (c - 1) * chunk, chunk)],
                    wsem[bp],
                )
        bl = (n_chunks - 1) & 1
        gh[bl].wait()
        wh[bl] = pltpu.async_copy(
            rows_v[bl],
            out_hbm.at[pl.ds(base + (n_chunks - 1) * chunk, chunk)],
            wsem[bl],
        )
        wh[bl ^ 1].wait()
        wh[bl].wait()

    return gather_k


@functools.lru_cache(maxsize=None)
def _sc_row_scatter(n_rows_out, n_idx, dim):
    info = plsc.get_sparse_core_info()
    nc, ns = info.num_cores, info.num_subcores
    nw = nc * ns
    assert n_idx % nw == 0
    b_per_w = n_idx // nw
    chunk = 64
    assert b_per_w % chunk == 0
    n_chunks = b_per_w // chunk
    assert n_chunks >= 2
    mesh = plsc.VectorSubcoreMesh(core_axis_name="c", subcore_axis_name="s")

    @functools.partial(
        pl.kernel,
        mesh=mesh,
        out_type=jax.ShapeDtypeStruct((n_rows_out, dim), jnp.float32),
        scratch_types=[
            pltpu.VMEM((chunk,), jnp.int32),
            pltpu.VMEM((chunk,), jnp.int32),
            pltpu.VMEM((chunk, dim), jnp.float32),
            pltpu.VMEM((chunk, dim), jnp.float32),
            pltpu.SemaphoreType.DMA,
            pltpu.SemaphoreType.DMA,
        ],
    )
    def scatter_k(rows_hbm, idx_hbm, out_hbm, i0, i1, r0, r1, s0, s1):
        wid = lax.axis_index("s") * nc + lax.axis_index("c")
        base = wid * b_per_w
        idx_v = [i0, i1]
        rows_v = [r0, r1]
        sems = [s0, s1]
        sh = [None, None]
        for c in range(n_chunks):
            b = c & 1
            off = base + c * chunk
            if c >= 2:
                sh[b].wait()
            pltpu.sync_copy(idx_hbm.at[pl.ds(off, chunk)], idx_v[b])
            pltpu.sync_copy(rows_hbm.at[pl.ds(off, chunk)], rows_v[b])
            sh[b] = pltpu.async_copy(rows_v[b], out_hbm.at[idx_v[b]], sems[b])
        bl = (n_chunks - 1) & 1
        sh[bl ^ 1].wait()
        sh[bl].wait()

    return scatter_k


GATE_BLK = 1024


def _lane_prefix_sum(v):
    li = lax.broadcasted_iota(jnp.int32, v.shape, 1)
    sh = 1
    while sh < v.shape[1]:
        rolled = pltpu.roll(v, sh, 1)
        v = v + jnp.where(li >= sh, rolled, 0)
        sh *= 2
    return v


def _gate_book_body(n_tiles, x_ref, gw_ref, gb_ref, inv_ref, te_ref,
                    eid_s, rank_s, carry_s, starts_s):
    i = pl.program_id(0)
    j = pl.program_id(1)
    e = NUM_EXPERT

    @pl.when(jnp.logical_and(i == 0, j == 0))
    def _init():
        carry_s[...] = jnp.zeros_like(carry_s)

    io8 = lax.broadcasted_iota(jnp.int32, (e, GATE_BLK), 0)

    @pl.when(i == 0)
    def _pass0():
        logits_t = lax.dot_general(
            gw_ref[...], x_ref[...], (((1,), (1,)), ((), ())),
            preferred_element_type=jnp.float32,
        ) + gb_ref[...].reshape(e, 1)
        m = jnp.max(logits_t, axis=0, keepdims=True)
        cand = jnp.where(logits_t >= m, io8, e)
        amin = jnp.min(cand, axis=0, keepdims=True)
        onehot = (io8 == amin).astype(jnp.int32)
        csum = _lane_prefix_sum(onehot)
        rank_local = jnp.sum(onehot * csum, axis=0, keepdims=True) - 1
        rank_glob = rank_local + jnp.sum(
            onehot * carry_s[...], axis=0, keepdims=True
        )
        eid = jnp.sum(onehot * io8, axis=0, keepdims=True)
        eid_s[pl.ds(j, 1), :] = eid
        rank_s[pl.ds(j, 1), :] = rank_glob
        carry_s[...] = carry_s[...] + csum[:, GATE_BLK - 1:GATE_BLK]

    @pl.when(i == 1)
    def _pass1():
        @pl.when(j == 0)
        def _starts():
            counts = carry_s[...]
            aligned = ((counts + TILE - 1) // TILE) * TILE
            fe = lax.broadcasted_iota(jnp.int32, (e, e), 1)
            ee = lax.broadcasted_iota(jnp.int32, (e, e), 0)
            al_t = jnp.transpose(jnp.broadcast_to(aligned.reshape(e, 1), (e, e)))
            starts = jnp.sum(jnp.where(fe < ee, al_t, 0), axis=1, keepdims=True)
            starts_s[...] = starts
            tb = lax.broadcasted_iota(jnp.int32, (e, n_tiles), 1) * TILE
            te = jnp.sum((tb >= starts).astype(jnp.int32), axis=0, keepdims=True) - 1
            te_ref[...] = jnp.clip(te, 0, e - 1)

        eid = eid_s[pl.ds(j, 1), :]
        onehot = (io8 == eid).astype(jnp.int32)
        start_tok = jnp.sum(onehot * starts_s[...], axis=0, keepdims=True)
        inv_ref[...] = (start_tok + rank_s[pl.ds(j, 1), :]).reshape(
            1, 1, GATE_BLK
        )


def _gate_book(x_flat, gate_w, gate_b, n_tiles):
    n = x_flat.shape[0]
    nb = n // GATE_BLK
    inv, te = pl.pallas_call(
        functools.partial(_gate_book_body, n_tiles),
        grid=(2, nb),
        in_specs=[
            pl.BlockSpec((GATE_BLK, EMBED_DIM), lambda i, j: ((1 - i) * j, 0)),
            pl.BlockSpec((NUM_EXPERT, EMBED_DIM), lambda i, j: (0, 0)),
            pl.BlockSpec((1, NUM_EXPERT), lambda i, j: (0, 0)),
        ],
        out_specs=[
            pl.BlockSpec((1, 1, GATE_BLK), lambda i, j: (j, 0, 0)),
            pl.BlockSpec((1, n_tiles), lambda i, j: (0, 0)),
        ],
        out_shape=[
            jax.ShapeDtypeStruct((nb, 1, GATE_BLK), jnp.int32),
            jax.ShapeDtypeStruct((1, n_tiles), jnp.int32),
        ],
        scratch_shapes=[
            pltpu.VMEM((nb, GATE_BLK), jnp.int32),
            pltpu.VMEM((nb, GATE_BLK), jnp.int32),
            pltpu.VMEM((NUM_EXPERT, 1), jnp.int32),
            pltpu.VMEM((NUM_EXPERT, 1), jnp.int32),
        ],
    )(x_flat, gate_w, gate_b.reshape(1, NUM_EXPERT))
    return inv.reshape(n), te.reshape(n_tiles)


def _moe_body(te_ref, x_ref, w1_ref, w2_ref, w3_ref, o_ref):
    e = te_ref[pl.program_id(0)]
    x = x_ref[...]
    h1 = lax.dot_general(
        x, w1_ref[e], (((1,), (1,)), ((), ())),
        preferred_element_type=jnp.float32,
    )
    h2 = lax.dot_general(
        x, w2_ref[e], (((1,), (1,)), ((), ())),
        preferred_element_type=jnp.float32,
    )
    h = h1 * jax.nn.sigmoid(h1) * h2
    o_ref[...] = lax.dot_general(
        h, w3_ref[e], (((1,), (1,)), ((), ())),
        preferred_element_type=jnp.float32,
    )


def _grouped_ffn(tile_expert, x_sorted, fc1_w, fc2_w, fc3_w):
    p = x_sorted.shape[0]
    grid = p // TILE
    return pl.pallas_call(
        _moe_body,
        grid_spec=pltpu.PrefetchScalarGridSpec(
            num_scalar_prefetch=1,
            grid=(grid,),
            in_specs=[
                pl.BlockSpec((TILE, EMBED_DIM), lambda g, te: (g, 0)),
                pl.BlockSpec(
                    (NUM_EXPERT, FFN_DIM, EMBED_DIM), lambda g, te: (0, 0, 0)
                ),
                pl.BlockSpec(
                    (NUM_EXPERT, FFN_DIM, EMBED_DIM), lambda g, te: (0, 0, 0)
                ),
                pl.BlockSpec(
                    (NUM_EXPERT, EMBED_DIM, FFN_DIM), lambda g, te: (0, 0, 0)
                ),
            ],
            out_specs=pl.BlockSpec((TILE, EMBED_DIM), lambda g, te: (g, 0)),
        ),
        out_shape=jax.ShapeDtypeStruct((p, EMBED_DIM), jnp.float32),
    )(tile_expert, x_sorted, fc1_w, fc2_w, fc3_w)


def kernel(x, gate_w, gate_b, fc1_w, fc2_w, fc3_w):
    b, s, d = x.shape
    n = b * s
    e = NUM_EXPERT
    p = n + e * TILE
    x_flat = x.reshape(n, d)

    inv_perm, tile_expert = _gate_book(x_flat, gate_w, gate_b, p // TILE)

    x_sorted = _sc_row_scatter(p, n, d)(x_flat, inv_perm)
    out_sorted = _grouped_ffn(tile_expert, x_sorted, fc1_w, fc2_w, fc3_w)
    out_flat = _sc_row_gather(p, n, d)(out_sorted, inv_perm)
    return out_flat.reshape(b, s, d)


# speedup vs baseline: 1.2275x; 1.2275x over previous
import functools

import jax
import jax.numpy as jnp
from jax import lax
from jax.experimental import pallas as pl
from jax.experimental.pallas import tpu as pltpu
from jax.experimental.pallas import tpu_sc as plsc

NUM_EXPERT = 8
EMBED_DIM = 768
FFN_DIM = 469
TILE = 256


@functools.lru_cache(maxsize=None)
def _sc_row_gather(n_rows_table, n_idx, dim):
    info = plsc.get_sparse_core_info()
    nc, ns = info.num_cores, info.num_subcores
    nw = nc * ns
    assert n_idx % nw == 0
    b_per_w = n_idx // nw
    chunk = 64
    assert b_per_w % chunk == 0
    n_chunks = b_per_w // chunk
    assert n_chunks >= 2
    mesh = plsc.VectorSubcoreMesh(core_axis_name="c", subcore_axis_name="s")

    @functools.partial(
        pl.kernel,
        mesh=mesh,
        out_type=jax.ShapeDtypeStruct((n_idx, dim), jnp.float32),
        scratch_types=[
            pltpu.VMEM((chunk,), jnp.int32),
            pltpu.VMEM((chunk,), jnp.int32),
            pltpu.VMEM((chunk, dim), jnp.float32),
            pltpu.VMEM((chunk, dim), jnp.float32),
            pltpu.SemaphoreType.DMA,
            pltpu.SemaphoreType.DMA,
            pltpu.SemaphoreType.DMA,
            pltpu.SemaphoreType.DMA,
        ],
    )
    def gather_k(table_hbm, idx_hbm, out_hbm, i0, i1, r0, r1, g0, g1, w0, w1):
        wid = lax.axis_index("s") * nc + lax.axis_index("c")
        base = wid * b_per_w
        idx_v = [i0, i1]
        rows_v = [r0, r1]
        gsem = [g0, g1]
        wsem = [w0, w1]
        gh = [None, None]
        wh = [None, None]
        for c in range(n_chunks):
            b = c & 1
            off = base + c * chunk
            if c >= 2:
                wh[b].wait()
            pltpu.sync_copy(idx_hbm.at[pl.ds(off, chunk)], idx_v[b])
            gh[b] = pltpu.async_copy(table_hbm.at[idx_v[b]], rows_v[b], gsem[b])
            if c >= 1:
                bp = (c - 1) & 1
                gh[bp].wait()
                wh[bp] = pltpu.async_copy(
                    rows_v[bp],
                    out_hbm.at[pl.ds(base + (c - 1) * chunk, chunk)],
                    wsem[bp],
                )
        bl = (n_chunks - 1) & 1
        gh[bl].wait()
        wh[bl] = pltpu.async_copy(
            rows_v[bl],
            out_hbm.at[pl.ds(base + (n_chunks - 1) * chunk, chunk)],
            wsem[bl],
        )
        wh[bl ^ 1].wait()
        wh[bl].wait()

    return gather_k


@functools.lru_cache(maxsize=None)
def _sc_row_scatter(n_rows_out, n_idx, dim):
    info = plsc.get_sparse_core_info()
    nc, ns = info.num_cores, info.num_subcores
    nw = nc * ns
    assert n_idx % nw == 0
    b_per_w = n_idx // nw
    chunk = 64
    assert b_per_w % chunk == 0
    n_chunks = b_per_w // chunk
    assert n_chunks >= 2
    mesh = plsc.VectorSubcoreMesh(core_axis_name="c", subcore_axis_name="s")

    @functools.partial(
        pl.kernel,
        mesh=mesh,
        out_type=jax.ShapeDtypeStruct((n_rows_out, dim), jnp.float32),
        scratch_types=[
            pltpu.VMEM((chunk,), jnp.int32),
            pltpu.VMEM((chunk,), jnp.int32),
            pltpu.VMEM((chunk, dim), jnp.float32),
            pltpu.VMEM((chunk, dim), jnp.float32),
            pltpu.SemaphoreType.DMA,
            pltpu.SemaphoreType.DMA,
        ],
    )
    def scatter_k(rows_hbm, idx_hbm, out_hbm, i0, i1, r0, r1, s0, s1):
        wid = lax.axis_index("s") * nc + lax.axis_index("c")
        base = wid * b_per_w
        idx_v = [i0, i1]
        rows_v = [r0, r1]
        sems = [s0, s1]
        sh = [None, None]
        for c in range(n_chunks):
            b = c & 1
            off = base + c * chunk
            if c >= 2:
                sh[b].wait()
            pltpu.sync_copy(idx_hbm.at[pl.ds(off, chunk)], idx_v[b])
            pltpu.sync_copy(rows_hbm.at[pl.ds(off, chunk)], rows_v[b])
            sh[b] = pltpu.async_copy(rows_v[b], out_hbm.at[idx_v[b]], sems[b])
        bl = (n_chunks - 1) & 1
        sh[bl ^ 1].wait()
        sh[bl].wait()

    return scatter_k


GATE_BLK = 1024


def _lane_prefix_sum(v):
    li = lax.broadcasted_iota(jnp.int32, v.shape, 1)
    sh = 1
    while sh < v.shape[1]:
        rolled = pltpu.roll(v, sh, 1)
        v = v + jnp.where(li >= sh, rolled, 0)
        sh *= 2
    return v


def _gate_book_body(n_tiles, x_ref, gw_ref, gb_ref, inv_ref, te_ref,
                    eid_s, rank_s, carry_s, starts_s):
    i = pl.program_id(0)
    j = pl.program_id(1)
    e = NUM_EXPERT

    @pl.when(jnp.logical_and(i == 0, j == 0))
    def _init():
        carry_s[...] = jnp.zeros_like(carry_s)

    io8 = lax.broadcasted_iota(jnp.int32, (e, GATE_BLK), 0)

    @pl.when(i == 0)
    def _pass0():
        logits_t = lax.dot_general(
            gw_ref[...], x_ref[...], (((1,), (1,)), ((), ())),
            preferred_element_type=jnp.float32,
        ) + gb_ref[...].reshape(e, 1)
        m = jnp.max(logits_t, axis=0, keepdims=True)
        cand = jnp.where(logits_t >= m, io8, e)
        amin = jnp.min(cand, axis=0, keepdims=True)
        onehot = (io8 == amin).astype(jnp.int32)
        csum = _lane_prefix_sum(onehot)
        rank_local = jnp.sum(onehot * csum, axis=0, keepdims=True) - 1
        rank_glob = rank_local + jnp.sum(
            onehot * carry_s[...], axis=0, keepdims=True
        )
        eid = jnp.sum(onehot * io8, axis=0, keepdims=True)
        eid_s[pl.ds(j, 1), :] = eid
        rank_s[pl.ds(j, 1), :] = rank_glob
        carry_s[...] = carry_s[...] + csum[:, GATE_BLK - 1:GATE_BLK]

    @pl.when(i == 1)
    def _pass1():
        @pl.when(j == 0)
        def _starts():
            counts = carry_s[...]
            aligned = ((counts + TILE - 1) // TILE) * TILE
            fe = lax.broadcasted_iota(jnp.int32, (e, e), 1)
            ee = lax.broadcasted_iota(jnp.int32, (e, e), 0)
            al_t = jnp.transpose(jnp.broadcast_to(aligned.reshape(e, 1), (e, e)))
            starts = jnp.sum(jnp.where(fe < ee, al_t, 0), axis=1, keepdims=True)
            starts_s[...] = starts
            tb = lax.broadcasted_iota(jnp.int32, (e, n_tiles), 1) * TILE
            te = jnp.sum((tb >= starts).astype(jnp.int32), axis=0, keepdims=True) - 1
            te_ref[...] = jnp.clip(te, 0, e - 1)

        eid = eid_s[pl.ds(j, 1), :]
        onehot = (io8 == eid).astype(jnp.int32)
        start_tok = jnp.sum(onehot * starts_s[...], axis=0, keepdims=True)
        inv_ref[...] = (start_tok + rank_s[pl.ds(j, 1), :]).reshape(
            1, 1, GATE_BLK
        )


def _gate_book(x_flat, gate_w, gate_b, n_tiles):
    n = x_flat.shape[0]
    nb = n // GATE_BLK
    inv, te = pl.pallas_call(
        functools.partial(_gate_book_body, n_tiles),
        grid=(2, nb),
        in_specs=[
            pl.BlockSpec((GATE_BLK, EMBED_DIM), lambda i, j: ((1 - i) * j, 0)),
            pl.BlockSpec((NUM_EXPERT, EMBED_DIM), lambda i, j: (0, 0)),
            pl.BlockSpec((1, NUM_EXPERT), lambda i, j: (0, 0)),
        ],
        out_specs=[
            pl.BlockSpec((1, 1, GATE_BLK), lambda i, j: (j, 0, 0)),
            pl.BlockSpec((1, n_tiles), lambda i, j: (0, 0)),
        ],
        out_shape=[
            jax.ShapeDtypeStruct((nb, 1, GATE_BLK), jnp.int32),
            jax.ShapeDtypeStruct((1, n_tiles), jnp.int32),
        ],
        scratch_shapes=[
            pltpu.VMEM((nb, GATE_BLK), jnp.int32),
            pltpu.VMEM((nb, GATE_BLK), jnp.int32),
            pltpu.VMEM((NUM_EXPERT, 1), jnp.int32),
            pltpu.VMEM((NUM_EXPERT, 1), jnp.int32),
        ],
    )(x_flat, gate_w, gate_b.reshape(1, NUM_EXPERT))
    return inv.reshape(n), te.reshape(n_tiles)


def _moe_body(te_ref, x_ref, w1_ref, w2_ref, w3_ref, o_ref):
    e = te_ref[pl.program_id(0)]
    x = x_ref[...]
    h1 = lax.dot_general(
        x, w1_ref[e], (((1,), (1,)), ((), ())),
        preferred_element_type=jnp.float32,
    )
    h2 = lax.dot_general(
        x, w2_ref[e], (((1,), (1,)), ((), ())),
        preferred_element_type=jnp.float32,
    )
    h = h1 * jax.nn.sigmoid(h1) * h2
    o_ref[...] = lax.dot_general(
        h, w3_ref[e], (((1,), (1,)), ((), ())),
        preferred_element_type=jnp.float32,
    )


def _grouped_ffn(tile_expert, x_sorted, fc1_w, fc2_w, fc3_w):
    p = x_sorted.shape[0]
    grid = p // TILE
    return pl.pallas_call(
        _moe_body,
        grid_spec=pltpu.PrefetchScalarGridSpec(
            num_scalar_prefetch=1,
            grid=(grid,),
            in_specs=[
                pl.BlockSpec((TILE, EMBED_DIM), lambda g, te: (g, 0)),
                pl.BlockSpec(
                    (NUM_EXPERT, FFN_DIM, EMBED_DIM), lambda g, te: (0, 0, 0)
                ),
                pl.BlockSpec(
                    (NUM_EXPERT, FFN_DIM, EMBED_DIM), lambda g, te: (0, 0, 0)
                ),
                pl.BlockSpec(
                    (NUM_EXPERT, EMBED_DIM, FFN_DIM), lambda g, te: (0, 0, 0)
                ),
            ],
            out_specs=pl.BlockSpec((TILE, EMBED_DIM), lambda g, te: (g, 0)),
        ),
        out_shape=jax.ShapeDtypeStruct((p, EMBED_DIM), jnp.float32),
    )(tile_expert, x_sorted, fc1_w, fc2_w, fc3_w)


def kernel(x, gate_w, gate_b, fc1_w, fc2_w, fc3_w):
    b, s, d = x.shape
    n = b * s
    e = NUM_EXPERT
    p = n + (e - 1) * TILE
    x_flat = x.reshape(n, d)

    inv_perm, tile_expert = _gate_book(x_flat, gate_w, gate_b, p // TILE)

    x_sorted = _sc_row_scatter(p, n, d)(x_flat, inv_perm)
    out_sorted = _grouped_ffn(tile_expert, x_sorted, fc1_w, fc2_w, fc3_w)
    out_flat = _sc_row_gather(p, n, d)(out_sorted, inv_perm)
    return out_flat.reshape(b, s, d)


# speedup vs baseline: 1.2539x; 1.0215x over previous
import functools

import jax
import jax.numpy as jnp
from jax import lax
from jax.experimental import pallas as pl
from jax.experimental.pallas import tpu as pltpu
from jax.experimental.pallas import tpu_sc as plsc

NUM_EXPERT = 8
EMBED_DIM = 768
FFN_DIM = 469
TILE = 256


@functools.lru_cache(maxsize=None)
def _sc_row_gather(n_rows_table, n_idx, dim):
    info = plsc.get_sparse_core_info()
    nc, ns = info.num_cores, info.num_subcores
    nw = nc * ns
    assert n_idx % nw == 0
    b_per_w = n_idx // nw
    chunk = 64
    assert b_per_w % chunk == 0
    n_chunks = b_per_w // chunk
    assert n_chunks >= 2
    mesh = plsc.VectorSubcoreMesh(core_axis_name="c", subcore_axis_name="s")

    @functools.partial(
        pl.kernel,
        mesh=mesh,
        out_type=jax.ShapeDtypeStruct((n_idx, dim), jnp.float32),
        scratch_types=[
            pltpu.VMEM((chunk,), jnp.int32),
            pltpu.VMEM((chunk,), jnp.int32),
            pltpu.VMEM((chunk, dim), jnp.float32),
            pltpu.VMEM((chunk, dim), jnp.float32),
            pltpu.SemaphoreType.DMA,
            pltpu.SemaphoreType.DMA,
            pltpu.SemaphoreType.DMA,
            pltpu.SemaphoreType.DMA,
        ],
    )
    def gather_k(table_hbm, idx_hbm, out_hbm, i0, i1, r0, r1, g0, g1, w0, w1):
        wid = lax.axis_index("s") * nc + lax.axis_index("c")
        base = wid * b_per_w
        idx_v = [i0, i1]
        rows_v = [r0, r1]
        gsem = [g0, g1]
        wsem = [w0, w1]
        gh = [None, None]
        wh = [None, None]
        for c in range(n_chunks):
            b = c & 1
            off = base + c * chunk
            if c >= 2:
                wh[b].wait()
            pltpu.sync_copy(idx_hbm.at[pl.ds(off, chunk)], idx_v[b])
            gh[b] = pltpu.async_copy(table_hbm.at[idx_v[b]], rows_v[b], gsem[b])
            if c >= 1:
                bp = (c - 1) & 1
                gh[bp].wait()
                wh[bp] = pltpu.async_copy(
                    rows_v[bp],
                    out_hbm.at[pl.ds(base + (c - 1) * chunk, chunk)],
                    wsem[bp],
                )
        bl = (n_chunks - 1) & 1
        gh[bl].wait()
        wh[bl] = pltpu.async_copy(
            rows_v[bl],
            out_hbm.at[pl.ds(base + (n_chunks - 1) * chunk, chunk)],
            wsem[bl],
        )
        wh[bl ^ 1].wait()
        wh[bl].wait()

    return gather_k


@functools.lru_cache(maxsize=None)
def _sc_row_scatter(n_rows_out, n_idx, dim):
    info = plsc.get_sparse_core_info()
    nc, ns = info.num_cores, info.num_subcores
    nw = nc * ns
    assert n_idx % nw == 0
    b_per_w = n_idx // nw
    chunk = 64
    assert b_per_w % chunk == 0
    n_chunks = b_per_w // chunk
    assert n_chunks >= 2
    mesh = plsc.VectorSubcoreMesh(core_axis_name="c", subcore_axis_name="s")

    @functools.partial(
        pl.kernel,
        mesh=mesh,
        out_type=jax.ShapeDtypeStruct((n_rows_out, dim), jnp.float32),
        scratch_types=[
            pltpu.VMEM((chunk,), jnp.int32),
            pltpu.VMEM((chunk,), jnp.int32),
            pltpu.VMEM((chunk, dim), jnp.float32),
            pltpu.VMEM((chunk, dim), jnp.float32),
            pltpu.SemaphoreType.DMA,
            pltpu.SemaphoreType.DMA,
        ],
    )
    def scatter_k(rows_hbm, idx_hbm, out_hbm, i0, i1, r0, r1, s0, s1):
        wid = lax.axis_index("s") * nc + lax.axis_index("c")
        base = wid * b_per_w
        idx_v = [i0, i1]
        rows_v = [r0, r1]
        sems = [s0, s1]
        sh = [None, None]
        for c in range(n_chunks):
            b = c & 1
            off = base + c * chunk
            if c >= 2:
                sh[b].wait()
            pltpu.sync_copy(idx_hbm.at[pl.ds(off, chunk)], idx_v[b])
            pltpu.sync_copy(rows_hbm.at[pl.ds(off, chunk)], rows_v[b])
            sh[b] = pltpu.async_copy(rows_v[b], out_hbm.at[idx_v[b]], sems[b])
        bl = (n_chunks - 1) & 1
        sh[bl ^ 1].wait()
        sh[bl].wait()

    return scatter_k


GATE_BLK = 2048


def _lane_prefix_sum(v):
    li = lax.broadcasted_iota(jnp.int32, v.shape, 1)
    sh = 1
    while sh < v.shape[1]:
        rolled = pltpu.roll(v, sh, 1)
        v = v + jnp.where(li >= sh, rolled, 0)
        sh *= 2
    return v


def _gate_book_body(n_tiles, x_ref, gw_ref, gb_ref, inv_ref, te_ref,
                    eid_s, rank_s, carry_s, starts_s):
    i = pl.program_id(0)
    j = pl.program_id(1)
    e = NUM_EXPERT

    @pl.when(jnp.logical_and(i == 0, j == 0))
    def _init():
        carry_s[...] = jnp.zeros_like(carry_s)

    io8 = lax.broadcasted_iota(jnp.int32, (e, GATE_BLK), 0)

    @pl.when(i == 0)
    def _pass0():
        logits_t = lax.dot_general(
            gw_ref[...], x_ref[...], (((1,), (1,)), ((), ())),
            preferred_element_type=jnp.float32,
        ) + gb_ref[...].reshape(e, 1)
        m = jnp.max(logits_t, axis=0, keepdims=True)
        cand = jnp.where(logits_t >= m, io8, e)
        amin = jnp.min(cand, axis=0, keepdims=True)
        onehot = (io8 == amin).astype(jnp.int32)
        csum = _lane_prefix_sum(onehot)
        rank_local = jnp.sum(onehot * csum, axis=0, keepdims=True) - 1
        rank_glob = rank_local + jnp.sum(
            onehot * carry_s[...], axis=0, keepdims=True
        )
        eid = jnp.sum(onehot * io8, axis=0, keepdims=True)
        eid_s[pl.ds(j, 1), :] = eid
        rank_s[pl.ds(j, 1), :] = rank_glob
        carry_s[...] = carry_s[...] + csum[:, GATE_BLK - 1:GATE_BLK]

    @pl.when(i == 1)
    def _pass1():
        @pl.when(j == 0)
        def _starts():
            counts = carry_s[...]
            aligned = ((counts + TILE - 1) // TILE) * TILE
            fe = lax.broadcasted_iota(jnp.int32, (e, e), 1)
            ee = lax.broadcasted_iota(jnp.int32, (e, e), 0)
            al_t = jnp.transpose(jnp.broadcast_to(aligned.reshape(e, 1), (e, e)))
            starts = jnp.sum(jnp.where(fe < ee, al_t, 0), axis=1, keepdims=True)
            starts_s[...] = starts
            tb = lax.broadcasted_iota(jnp.int32, (e, n_tiles), 1) * TILE
            te = jnp.sum((tb >= starts).astype(jnp.int32), axis=0, keepdims=True) - 1
            te_ref[...] = jnp.clip(te, 0, e - 1)

        eid = eid_s[pl.ds(j, 1), :]
        onehot = (io8 == eid).astype(jnp.int32)
        start_tok = jnp.sum(onehot * starts_s[...], axis=0, keepdims=True)
        inv_ref[...] = (start_tok + rank_s[pl.ds(j, 1), :]).reshape(
            1, 1, GATE_BLK
        )


def _gate_book(x_flat, gate_w, gate_b, n_tiles):
    n = x_flat.shape[0]
    nb = n // GATE_BLK
    inv, te = pl.pallas_call(
        functools.partial(_gate_book_body, n_tiles),
        grid=(2, nb),
        in_specs=[
            pl.BlockSpec((GATE_BLK, EMBED_DIM), lambda i, j: ((1 - i) * j, 0)),
            pl.BlockSpec((NUM_EXPERT, EMBED_DIM), lambda i, j: (0, 0)),
            pl.BlockSpec((1, NUM_EXPERT), lambda i, j: (0, 0)),
        ],
        out_specs=[
            pl.BlockSpec((1, 1, GATE_BLK), lambda i, j: (j, 0, 0)),
            pl.BlockSpec((1, n_tiles), lambda i, j: (0, 0)),
        ],
        out_shape=[
            jax.ShapeDtypeStruct((nb, 1, GATE_BLK), jnp.int32),
            jax.ShapeDtypeStruct((1, n_tiles), jnp.int32),
        ],
        scratch_shapes=[
            pltpu.VMEM((nb, GATE_BLK), jnp.int32),
            pltpu.VMEM((nb, GATE_BLK), jnp.int32),
            pltpu.VMEM((NUM_EXPERT, 1), jnp.int32),
            pltpu.VMEM((NUM_EXPERT, 1), jnp.int32),
        ],
    )(x_flat, gate_w, gate_b.reshape(1, NUM_EXPERT))
    return inv.reshape(n), te.reshape(n_tiles)


def _moe_body(te_ref, x_ref, w1_ref, w2_ref, w3_ref, o_ref):
    e = te_ref[pl.program_id(0)]
    x = x_ref[...]
    h1 = lax.dot_general(
        x, w1_ref[e], (((1,), (1,)), ((), ())),
        preferred_element_type=jnp.float32,
    )
    h2 = lax.dot_general(
        x, w2_ref[e], (((1,), (1,)), ((), ())),
        preferred_element_type=jnp.float32,
    )
    h = h1 * jax.nn.sigmoid(h1) * h2
    o_ref[...] = lax.dot_general(
        h, w3_ref[e], (((1,), (1,)), ((), ())),
        preferred_element_type=jnp.float32,
    )


def _grouped_ffn(tile_expert, x_sorted, fc1_w, fc2_w, fc3_w):
    p = x_sorted.shape[0]
    grid = p // TILE
    return pl.pallas_call(
        _moe_body,
        grid_spec=pltpu.PrefetchScalarGridSpec(
            num_scalar_prefetch=1,
            grid=(grid,),
            in_specs=[
                pl.BlockSpec((TILE, EMBED_DIM), lambda g, te: (g, 0)),
                pl.BlockSpec(
                    (NUM_EXPERT, FFN_DIM, EMBED_DIM), lambda g, te: (0, 0, 0)
                ),
                pl.BlockSpec(
                    (NUM_EXPERT, FFN_DIM, EMBED_DIM), lambda g, te: (0, 0, 0)
                ),
                pl.BlockSpec(
                    (NUM_EXPERT, EMBED_DIM, FFN_DIM), lambda g, te: (0, 0, 0)
                ),
            ],
            out_specs=pl.BlockSpec((TILE, EMBED_DIM), lambda g, te: (g, 0)),
        ),
        out_shape=jax.ShapeDtypeStruct((p, EMBED_DIM), jnp.float32),
    )(tile_expert, x_sorted, fc1_w, fc2_w, fc3_w)


def kernel(x, gate_w, gate_b, fc1_w, fc2_w, fc3_w):
    b, s, d = x.shape
    n = b * s
    e = NUM_EXPERT
    p = n + (e - 1) * TILE
    x_flat = x.reshape(n, d)

    inv_perm, tile_expert = _gate_book(x_flat, gate_w, gate_b, p // TILE)

    x_sorted = _sc_row_scatter(p, n, d)(x_flat, inv_perm)
    out_sorted = _grouped_ffn(tile_expert, x_sorted, fc1_w, fc2_w, fc3_w)
    out_flat = _sc_row_gather(p, n, d)(out_sorted, inv_perm)
    return out_flat.reshape(b, s, d)
